# Initial kernel scaffold; baseline (speedup 1.0000x reference)
import jax
import jax.numpy as jnp
from jax import lax
from jax.experimental import pallas as pl
from jax.experimental.pallas import tpu as pltpu
from jax.experimental.pallas import tpu_sc as plsc

D_MODEL = 64
NC, NS = 2, 16
NW = NC * NS
CHUNK = 256
IPG = 128
NG = 2 * CHUNK // IPG


def _tec_body(x_hbm, tab_hbm, out_hbm, x_v, idx_v, rows_v, out_v, sem):
    wid = lax.axis_index("s") * NC + lax.axis_index("c")
    n_pos = x_hbm.shape[0] // 2
    per_w = n_pos // NW
    n_chunks = per_w // CHUNK
    base = wid * per_w

    @pl.loop(0, n_chunks)
    def _chunk(g):
        pbase = base + g * CHUNK
        pltpu.sync_copy(x_hbm.at[pl.ds(pbase * 2, 2 * CHUNK)], x_v)
        for i in range(2 * CHUNK // 16):
            xv = x_v[pl.ds(i * 16, 16)]
            iv = (xv * 5000.0 + 5000.0).astype(jnp.int32)
            idx_v[i // (IPG // 16), pl.ds((i % (IPG // 16)) * 16, 16)] = iv
        copies = [
            pltpu.async_copy(
                tab_hbm.at[idx_v.at[j]],
                rows_v.at[pl.ds(j * IPG, IPG)],
                sem,
            )
            for j in range(NG)
        ]
        for c in copies:
            c.wait()

        @pl.loop(0, CHUNK)
        def _add(p):
            for d in range(D_MODEL // 16):
                sl = pl.ds(d * 16, 16)
                out_v[p, sl] = rows_v[2 * p, sl] + rows_v[2 * p + 1, sl]

        pltpu.sync_copy(out_v, out_hbm.at[pl.ds(pbase, CHUNK)])


def kernel(x, pos_enc):
    b, t, _ = x.shape
    n_pos = b * t
    x_flat = x.reshape(n_pos * 2)

    mesh = plsc.VectorSubcoreMesh(
        core_axis_name="c", subcore_axis_name="s", num_cores=NC, num_subcores=NS
    )
    run = pl.kernel(
        _tec_body,
        out_type=jax.ShapeDtypeStruct((n_pos, D_MODEL), jnp.float32),
        mesh=mesh,
        scratch_types=[
            pltpu.VMEM((2 * CHUNK,), jnp.float32),
            pltpu.VMEM((NG, IPG), jnp.int32),
            pltpu.VMEM((2 * CHUNK, D_MODEL), jnp.float32),
            pltpu.VMEM((CHUNK, D_MODEL), jnp.float32),
            pltpu.SemaphoreType.DMA,
        ],
    )
    out = run(x_flat, pos_enc)
    return out.reshape(b, t, D_MODEL)


# speedup vs baseline: 2.6382x; 2.6382x over previous
import jax
import jax.numpy as jnp
from jax import lax
from jax.experimental import pallas as pl
from jax.experimental.pallas import tpu as pltpu
from jax.experimental.pallas import tpu_sc as plsc

D_MODEL = 64
NC, NS = 2, 16
NW = NC * NS
CHUNK = 256
IPG = 128
NG = 2 * CHUNK // IPG


def _tec_body(x_hbm, tab_hbm, out_hbm, x_v, idx_v, rows_v, out_v, sem):
    wid = lax.axis_index("s") * NC + lax.axis_index("c")
    n_pos = x_hbm.shape[0] // 2
    per_w = n_pos // NW
    n_chunks = per_w // CHUNK
    base = wid * per_w

    @pl.loop(0, n_chunks)
    def _chunk(g):
        pbase = base + g * CHUNK
        pltpu.sync_copy(x_hbm.at[pl.ds(pbase * 2, 2 * CHUNK)], x_v)
        for i in range(2 * CHUNK // 16):
            xv = x_v[pl.ds(i * 16, 16)]
            iv = (xv * 5000.0 + 5000.0).astype(jnp.int32)
            idx_v[i // (IPG // 16), pl.ds((i % (IPG // 16)) * 16, 16)] = iv
        copies = [
            pltpu.async_copy(
                tab_hbm.at[idx_v.at[j]],
                rows_v.at[pl.ds(j * IPG, IPG)],
                sem,
            )
            for j in range(NG)
        ]
        for c in copies:
            c.wait()

        @pl.loop(0, CHUNK)
        def _add(p):
            for d in range(D_MODEL // 16):
                sl = pl.ds(d * 16, 16)
                out_v[p, sl] = rows_v[2 * p, sl] + rows_v[2 * p + 1, sl]

        pltpu.sync_copy(out_v, out_hbm.at[pl.ds(pbase, CHUNK)])


def kernel(x, pos_enc):
    b, t, _ = x.shape
    n_pos = b * t
    x_flat = x.reshape(n_pos * 2)

    mesh = plsc.VectorSubcoreMesh(
        core_axis_name="c", subcore_axis_name="s", num_cores=NC, num_subcores=NS
    )
    run = pl.kernel(
        _tec_body,
        out_type=jax.ShapeDtypeStruct((n_pos, D_MODEL), jnp.float32),
        mesh=mesh,
        scratch_types=[
            pltpu.VMEM((2 * CHUNK,), jnp.float32),
            pltpu.VMEM((NG, IPG), jnp.int32),
            pltpu.VMEM((2 * CHUNK, D_MODEL), jnp.float32),
            pltpu.VMEM((CHUNK, D_MODEL), jnp.float32),
            pltpu.SemaphoreType.DMA,
        ],
        compiler_params=pltpu.CompilerParams(use_tc_tiling_on_sc=False),
    )
    out = run(x_flat, pos_enc)
    return out.reshape(b, t, D_MODEL)


# speedup vs baseline: 3.1064x; 1.1775x over previous
import jax
import jax.numpy as jnp
from jax import lax
from jax.experimental import pallas as pl
from jax.experimental.pallas import tpu as pltpu
from jax.experimental.pallas import tpu_sc as plsc

D_MODEL = 64
NC, NS = 2, 16
NW = NC * NS
CHUNK = 256
IPG = 128
NG = 2 * CHUNK // IPG


def _tec_body(x_hbm, tab_hbm, out_hbm,
              x_vA, x_vB, idx_vA, idx_vB, rows_vA, rows_vB, out_vA, out_vB,
              xsemA, xsemB, gsemA, gsemB, osemA, osemB):
    wid = lax.axis_index("s") * NC + lax.axis_index("c")
    n_pos = x_hbm.shape[0] // 2
    per_w = n_pos // NW
    n_chunks = per_w // CHUNK
    base = wid * per_w

    def x_copy(g, x_v, xsem):
        return pltpu.make_async_copy(
            x_hbm.at[pl.ds((base + g * CHUNK) * 2, 2 * CHUNK)], x_v, xsem)

    def gather_copies(idx_v, rows_v, gsem):
        return [
            pltpu.make_async_copy(
                tab_hbm.at[idx_v.at[j]], rows_v.at[pl.ds(j * IPG, IPG)], gsem)
            for j in range(NG)
        ]

    def out_copy(g, out_v, osem):
        return pltpu.make_async_copy(
            out_v, out_hbm.at[pl.ds(base + g * CHUNK, CHUNK)], osem)

    def compute_idx(x_v, idx_v):
        for i in range(2 * CHUNK // 16):
            xv = x_v[pl.ds(i * 16, 16)]
            iv = (xv * 5000.0 + 5000.0).astype(jnp.int32)
            idx_v[i // (IPG // 16), pl.ds((i % (IPG // 16)) * 16, 16)] = iv

    def fire_gathers(idx_v, rows_v, gsem):
        for c in gather_copies(idx_v, rows_v, gsem):
            c.start()

    def drain_gathers(idx_v, rows_v, gsem):
        for c in gather_copies(idx_v, rows_v, gsem):
            c.wait()

    def add_pairs(rows_v, out_v):
        @pl.loop(0, CHUNK, unroll=8)
        def _add(p):
            for d in range(D_MODEL // 16):
                sl = pl.ds(d * 16, 16)
                out_v[p, sl] = rows_v[2 * p, sl] + rows_v[2 * p + 1, sl]

    x_copy(0, x_vA, xsemA).start()
    x_copy(1, x_vB, xsemB).start()
    x_copy(0, x_vA, xsemA).wait()
    compute_idx(x_vA, idx_vA)
    fire_gathers(idx_vA, rows_vA, gsemA)

    @pl.loop(0, n_chunks // 2)
    def _iter(k):
        a = 2 * k
        x_copy(a + 1, x_vB, xsemB).wait()
        compute_idx(x_vB, idx_vB)
        fire_gathers(idx_vB, rows_vB, gsemB)

        @pl.when(a + 2 < n_chunks)
        def _():
            x_copy(a + 2, x_vA, xsemA).start()

        drain_gathers(idx_vA, rows_vA, gsemA)

        @pl.when(k >= 1)
        def _():
            out_copy(a - 2, out_vA, osemA).wait()

        add_pairs(rows_vA, out_vA)
        out_copy(a, out_vA, osemA).start()

        @pl.when(a + 2 < n_chunks)
        def _():
            x_copy(a + 2, x_vA, xsemA).wait()
            compute_idx(x_vA, idx_vA)
            fire_gathers(idx_vA, rows_vA, gsemA)
            x_copy(a + 3, x_vB, xsemB).start()

        drain_gathers(idx_vB, rows_vB, gsemB)

        @pl.when(k >= 1)
        def _():
            out_copy(a - 1, out_vB, osemB).wait()

        add_pairs(rows_vB, out_vB)
        out_copy(a + 1, out_vB, osemB).start()

    out_copy(n_chunks - 2, out_vA, osemA).wait()
    out_copy(n_chunks - 1, out_vB, osemB).wait()


def kernel(x, pos_enc):
    b, t, _ = x.shape
    n_pos = b * t
    x_flat = x.reshape(n_pos * 2)

    mesh = plsc.VectorSubcoreMesh(
        core_axis_name="c", subcore_axis_name="s", num_cores=NC, num_subcores=NS
    )
    run = pl.kernel(
        _tec_body,
        out_type=jax.ShapeDtypeStruct((n_pos, D_MODEL), jnp.float32),
        mesh=mesh,
        scratch_types=[
            pltpu.VMEM((2 * CHUNK,), jnp.float32),
            pltpu.VMEM((2 * CHUNK,), jnp.float32),
            pltpu.VMEM((NG, IPG), jnp.int32),
            pltpu.VMEM((NG, IPG), jnp.int32),
            pltpu.VMEM((2 * CHUNK, D_MODEL), jnp.float32),
            pltpu.VMEM((2 * CHUNK, D_MODEL), jnp.float32),
            pltpu.VMEM((CHUNK, D_MODEL), jnp.float32),
            pltpu.VMEM((CHUNK, D_MODEL), jnp.float32),
            pltpu.SemaphoreType.DMA,
            pltpu.SemaphoreType.DMA,
            pltpu.SemaphoreType.DMA,
            pltpu.SemaphoreType.DMA,
            pltpu.SemaphoreType.DMA,
            pltpu.SemaphoreType.DMA,
        ],
        compiler_params=pltpu.CompilerParams(use_tc_tiling_on_sc=False),
    )
    out = run(x_flat, pos_enc)
    return out.reshape(b, t, D_MODEL)
